# Initial kernel scaffold; baseline (speedup 1.0000x reference)
import jax
import jax.numpy as jnp
from jax.experimental import pallas as pl


def kernel(x, edge_index, We1, be1, We2, be2, Wfc, bfc, Wg1, bg1, Wg2, bg2, Wc, bc):
    raise NotImplementedError("write your pallas kernel here")


# speedup vs baseline: 4.8847x; 4.8847x over previous
import functools

import jax
import jax.numpy as jnp
from jax import lax
from jax.experimental import pallas as pl
from jax.experimental.pallas import tpu as pltpu
from jax.experimental.pallas import tpu_sc as plsc

_N = 10000
_E = 160000
_NS = 16
_EP = _E // _NS
_BK = 80
_NCK = 5
_JB = 25
_NP = 10240
_RPS = _NP // _NS
_ZR = 32
_NZ = _RPS // _ZR
_SR = _NP // 128
_BM = 1024
_HI = jax.lax.Precision.HIGHEST

_MESH = plsc.VectorSubcoreMesh(core_axis_name="c", subcore_axis_name="s")


def _sc_degrees(er):
    @functools.partial(
        pl.kernel,
        out_type=jax.ShapeDtypeStruct((2, _NP, 128), jnp.float32),
        mesh=_MESH,
        scratch_types=[
            pltpu.VMEM((_JB, _BK), jnp.int32),
            pltpu.VMEM((_BK, 128), jnp.float32),
            pltpu.VMEM((_ZR, 128), jnp.float32),
            pltpu.VMEM_SHARED((_NP, 128), jnp.float32),
        ],
    )
    def k(er_hbm, deg_hbm, idx_v, ones_v, zrow_v, acc):
        c = lax.axis_index("c")
        s = lax.axis_index("s")
        one = jnp.ones((16,), jnp.float32)
        zero = jnp.zeros((16,), jnp.float32)

        def fill(i, carry):
            def f2(m, carry2):
                ones_v[i, pl.ds(m * 16, 16)] = one
                return carry2

            lax.fori_loop(0, 8, f2, 0)
            return carry

        lax.fori_loop(0, _BK, fill, 0)

        def fzero(i, carry):
            def fz2(m, carry2):
                zrow_v[i, pl.ds(m * 16, 16)] = zero
                return carry2

            lax.fori_loop(0, 8, fz2, 0)
            return carry

        lax.fori_loop(0, _ZR, fzero, 0)

        for t in range(_NZ):
            pltpu.sync_copy(zrow_v, acc.at[pl.ds(s * _RPS + t * _ZR, _ZR)])
        plsc.subcore_barrier()

        def chunk(g, carry):
            pltpu.sync_copy(er_hbm.at[c, s, g], idx_v)

            def body(j, carry2):
                pltpu.sync_copy(ones_v, acc.at[idx_v.at[j]], add=True)
                return carry2

            lax.fori_loop(0, _JB, body, 0)
            return carry

        lax.fori_loop(0, _NCK, chunk, 0)
        plsc.subcore_barrier()
        pltpu.sync_copy(acc.at[pl.ds(s * _RPS, _RPS)],
                        deg_hbm.at[c, pl.ds(s * _RPS, _RPS)])

    return k(er)


def _sc_agg128(tab, er):
    @functools.partial(
        pl.kernel,
        out_type=jax.ShapeDtypeStruct((2, _NP, 128), jnp.float32),
        mesh=_MESH,
        scratch_types=[
            pltpu.VMEM((_JB, _BK), jnp.int32),
            pltpu.VMEM((_JB, _BK), jnp.int32),
            pltpu.VMEM((_BK, 128), jnp.float32),
            pltpu.VMEM((_ZR, 128), jnp.float32),
            pltpu.VMEM_SHARED((_NP, 128), jnp.float32),
            pltpu.SemaphoreType.DMA,
        ],
    )
    def k(tab_hbm, er_hbm, out_hbm, src_v, dst_v, rows_v, zrow_v, acc, sem):
        c = lax.axis_index("c")
        s = lax.axis_index("s")
        zero = jnp.zeros((16,), jnp.float32)

        def fzero(i, carry):
            def fz2(m, carry2):
                zrow_v[i, pl.ds(m * 16, 16)] = zero
                return carry2

            lax.fori_loop(0, 8, fz2, 0)
            return carry

        lax.fori_loop(0, _ZR, fzero, 0)
        for t in range(_NZ):
            pltpu.sync_copy(zrow_v, acc.at[pl.ds(s * _RPS + t * _ZR, _ZR)])
        plsc.subcore_barrier()

        def chunk(g, carry):
            pltpu.sync_copy(er_hbm.at[0, s, g], src_v)
            pltpu.sync_copy(er_hbm.at[1, s, g], dst_v)

            def body(j, carry2):
                pltpu.async_copy(tab_hbm.at[c].at[src_v.at[j]], rows_v,
                                 sem).wait()
                pltpu.sync_copy(rows_v, acc.at[dst_v.at[j]], add=True)
                return carry2

            lax.fori_loop(0, _JB, body, 0)
            return carry

        lax.fori_loop(0, _NCK, chunk, 0)
        plsc.subcore_barrier()
        pltpu.sync_copy(acc.at[pl.ds(s * _RPS, _RPS)],
                        out_hbm.at[c, pl.ds(s * _RPS, _RPS)])

    return k(tab, er)


def _sc_scol(st1, er1):
    @functools.partial(
        pl.kernel,
        out_type=jax.ShapeDtypeStruct((_NS * _NP,), jnp.float32),
        mesh=_MESH,
        compiler_params=pltpu.CompilerParams(needs_layout_passes=False),
        scratch_types=[
            pltpu.VMEM((_EP,), jnp.int32),
            pltpu.VMEM((_EP,), jnp.int32),
            pltpu.VMEM((_NP,), jnp.float32),
            pltpu.VMEM((_NP,), jnp.float32),
        ],
    )
    def k(st_hbm, er_hbm, outs_hbm, src_v, dst_v, stcol_v, sacc_v):
        c = lax.axis_index("c")
        s = lax.axis_index("s")
        zero = jnp.zeros((16,), jnp.float32)

        @pl.when(c == 0)
        def _():
            pltpu.sync_copy(st_hbm, stcol_v)
            pltpu.sync_copy(er_hbm.at[pl.ds(s * _EP, _EP)], src_v)
            pltpu.sync_copy(er_hbm.at[pl.ds(_E + s * _EP, _EP)], dst_v)

            def fzs(i, carry):
                sacc_v[pl.ds(i * 16, 16)] = zero
                return carry

            lax.fori_loop(0, _NP // 16, fzs, 0)

            def body(t, carry):
                i16 = src_v[pl.ds(t * 16, 16)]
                d16 = dst_v[pl.ds(t * 16, 16)]
                v16 = plsc.load_gather(stcol_v, [i16])
                plsc.addupdate_scatter(sacc_v, [d16], v16)
                return carry

            lax.fori_loop(0, _EP // 16, body, 0)
            pltpu.sync_copy(sacc_v, outs_hbm.at[pl.ds(s * _NP, _NP)])

    return k(st1, er1)


def _tc_prep_body(x_ref, dego_ref, out_ref):
    d = dego_ref[:, 0:1]
    ns = jnp.where(d > 0, lax.rsqrt(d), 0.0)
    xn = x_ref[...] * ns
    out_ref[0] = xn[:, :128]
    out_ref[1] = xn[:, 128:]


def _tc_prep(x, dego):
    bn = 1000
    return pl.pallas_call(
        _tc_prep_body,
        grid=(_N // bn,),
        in_specs=[
            pl.BlockSpec((bn, 256), lambda i: (i, 0)),
            pl.BlockSpec((bn, 128), lambda i: (i, 0)),
        ],
        out_specs=pl.BlockSpec((2, bn, 128), lambda i: (0, i, 0)),
        out_shape=jax.ShapeDtypeStruct((2, _N, 128), jnp.float32),
    )(x, dego)


def _tc_mid_body(agg1_ref, degi_ref, dego_ref, We1_ref, be1_ref, Wg1_ref,
                 bg1_ref, We2_ref, Wfc_ref, ct_ref, st_ref):
    a = jnp.concatenate([agg1_ref[0], agg1_ref[1]], axis=1)
    di = degi_ref[:, 0:1]
    nd = jnp.where(di > 0, lax.rsqrt(di), 0.0)
    do = dego_ref[:, 0:1]
    no = jnp.where(do > 0, lax.rsqrt(do), 0.0)
    a = a * nd
    h1 = jax.nn.relu(jnp.dot(a, We1_ref[...], precision=_HI) + be1_ref[...])
    z1 = jax.nn.relu(jnp.dot(a, Wg1_ref[...], precision=_HI) + bg1_ref[...])
    wv = jnp.dot(We2_ref[...], Wfc_ref[...], precision=_HI)
    sc = jnp.dot(h1, wv, precision=_HI)
    z1n = z1 * no
    ct_ref[0] = z1n[:, :128]
    ct_ref[1] = z1n[:, 128:]
    st_ref[...] = (sc[:, 0:1] * no).reshape(_BM // 128, 128)


def _tc_mid(agg1, degi, dego, We1, be1r, Wg1, bg1r, We2, Wfc):
    full = lambda shape: pl.BlockSpec(shape, lambda i: tuple(0 for _ in shape))
    return pl.pallas_call(
        _tc_mid_body,
        grid=(_NP // _BM,),
        in_specs=[
            pl.BlockSpec((2, _BM, 128), lambda i: (0, i, 0)),
            pl.BlockSpec((_BM, 128), lambda i: (i, 0)),
            pl.BlockSpec((_BM, 128), lambda i: (i, 0)),
            full((256, 256)),
            full((1, 256)),
            full((256, 256)),
            full((1, 256)),
            full((256, 256)),
            full((256, 1)),
        ],
        out_specs=(
            pl.BlockSpec((2, _BM, 128), lambda i: (0, i, 0)),
            pl.BlockSpec((_BM // 128, 128), lambda i: (i, 0)),
        ),
        out_shape=(
            jax.ShapeDtypeStruct((2, _NP, 128), jnp.float32),
            jax.ShapeDtypeStruct((_SR, 128), jnp.float32),
        ),
    )(agg1, degi, dego, We1, be1r, Wg1, bg1r, We2, Wfc)


def _tc_final_body(agg2_ref, part_ref, degi_ref, degf_ref, Wg2_ref, bg2_ref,
                   Wc_ref, bc_ref, be2_ref, Wfc_ref, bfc_ref, y_ref, sf_ref,
                   z_ref):
    q = jnp.concatenate([agg2_ref[0], agg2_ref[1]], axis=1)
    di = degi_ref[:, 0:1]
    nd = jnp.where(di > 0, lax.rsqrt(di), 0.0)
    z = jnp.dot(q * nd, Wg2_ref[...], precision=_HI) + bg2_ref[...]
    y = jnp.dot(z, Wc_ref[...], precision=_HI) + bc_ref[...]
    sconst = jnp.dot(be2_ref[...], Wfc_ref[...], precision=_HI) + bfc_ref[...]
    df = degf_ref[...]
    ndf = jnp.where(df > 0, lax.rsqrt(df), 0.0)
    sagg = jnp.sum(part_ref[...], axis=0)
    sf_ref[...] = sagg * ndf + sconst
    y_ref[...] = y
    z_ref[...] = z


def _tc_final(agg2, parts, degi, degf, Wg2, bg2r, Wc, bcr, be2r, Wfc, bfcr):
    full = lambda shape: pl.BlockSpec(shape, lambda i: tuple(0 for _ in shape))
    return pl.pallas_call(
        _tc_final_body,
        grid=(_NP // _BM,),
        in_specs=[
            pl.BlockSpec((2, _BM, 128), lambda i: (0, i, 0)),
            pl.BlockSpec((_NS, _BM // 128, 128), lambda i: (0, i, 0)),
            pl.BlockSpec((_BM, 128), lambda i: (i, 0)),
            pl.BlockSpec((_BM // 128, 128), lambda i: (i, 0)),
            full((256, 256)),
            full((1, 256)),
            full((256, 1)),
            full((1, 1)),
            full((1, 256)),
            full((256, 1)),
            full((1, 1)),
        ],
        out_specs=(
            pl.BlockSpec((_BM, 1), lambda i: (i, 0)),
            pl.BlockSpec((_BM // 128, 128), lambda i: (i, 0)),
            pl.BlockSpec((_BM, 256), lambda i: (i, 0)),
        ),
        out_shape=(
            jax.ShapeDtypeStruct((_N, 1), jnp.float32),
            jax.ShapeDtypeStruct((_NP // 128, 128), jnp.float32),
            jax.ShapeDtypeStruct((_N, 256), jnp.float32),
        ),
    )(agg2, parts, degi, degf, Wg2, bg2r, Wc, bcr, be2r, Wfc, bfcr)


def kernel(x, edge_index, We1, be1, We2, be2, Wfc, bfc, Wg1, bg1, Wg2, bg2,
           Wc, bc):
    er = edge_index.reshape(2, _NS, _NCK, _JB, _BK)
    deg = _sc_degrees(er)
    dego = deg[0]
    degi = deg[1]
    xnt = _tc_prep(x, dego)
    agg1 = _sc_agg128(xnt, er)
    ct, st = _tc_mid(agg1, degi, dego, We1, be1.reshape(1, -1), Wg1,
                     bg1.reshape(1, -1), We2, Wfc)
    agg2 = _sc_agg128(ct, er)
    sparts = _sc_scol(st.reshape(-1), edge_index.reshape(-1))
    parts = sparts.reshape(_NS, _NP // 128, 128)
    degf = degi[:, 0].reshape(_NP // 128, 128)
    y, sf, z = _tc_final(agg2, parts, degi, degf, Wg2, bg2.reshape(1, -1), Wc,
                         bc.reshape(1, 1), be2.reshape(1, -1), Wfc,
                         bfc.reshape(1, 1))
    s = sf.reshape(-1)[:_N].reshape(_N, 1)
    return (y, s, z)


# speedup vs baseline: 5.7792x; 1.1831x over previous
import functools

import jax
import jax.numpy as jnp
from jax import lax
from jax.experimental import pallas as pl
from jax.experimental.pallas import tpu as pltpu
from jax.experimental.pallas import tpu_sc as plsc

_N = 10000
_E = 160000
_NS = 16
_EP = _E // _NS
_BK = 80
_NCK = 5
_JB = 25
_NP = 10240
_RPS = _NP // _NS
_ZR = 32
_NZ = _RPS // _ZR
_SR = _NP // 128
_BM = 1024
_HI = jax.lax.Precision.HIGHEST

_MESH = plsc.VectorSubcoreMesh(core_axis_name="c", subcore_axis_name="s")


def _sc_degrees(er):
    @functools.partial(
        pl.kernel,
        out_type=jax.ShapeDtypeStruct((2, _NP, 128), jnp.float32),
        mesh=_MESH,
        scratch_types=[
            pltpu.VMEM((_JB, _BK), jnp.int32),
            pltpu.VMEM((_BK, 128), jnp.float32),
            pltpu.VMEM((_ZR, 128), jnp.float32),
            pltpu.VMEM_SHARED((_NP, 128), jnp.float32),
        ],
    )
    def k(er_hbm, deg_hbm, idx_v, ones_v, zrow_v, acc):
        c = lax.axis_index("c")
        s = lax.axis_index("s")
        one = jnp.ones((16,), jnp.float32)
        zero = jnp.zeros((16,), jnp.float32)

        def fill(i, carry):
            def f2(m, carry2):
                ones_v[i, pl.ds(m * 16, 16)] = one
                return carry2

            lax.fori_loop(0, 8, f2, 0)
            return carry

        lax.fori_loop(0, _BK, fill, 0)

        def fzero(i, carry):
            def fz2(m, carry2):
                zrow_v[i, pl.ds(m * 16, 16)] = zero
                return carry2

            lax.fori_loop(0, 8, fz2, 0)
            return carry

        lax.fori_loop(0, _ZR, fzero, 0)

        for t in range(_NZ):
            pltpu.sync_copy(zrow_v, acc.at[pl.ds(s * _RPS + t * _ZR, _ZR)])
        plsc.subcore_barrier()

        def chunk(g, carry):
            pltpu.sync_copy(er_hbm.at[c, s, g], idx_v)

            def body(j, carry2):
                pltpu.sync_copy(ones_v, acc.at[idx_v.at[j]], add=True)
                return carry2

            lax.fori_loop(0, _JB, body, 0)
            return carry

        lax.fori_loop(0, _NCK, chunk, 0)
        plsc.subcore_barrier()
        pltpu.sync_copy(acc.at[pl.ds(s * _RPS, _RPS)],
                        deg_hbm.at[c, pl.ds(s * _RPS, _RPS)])

    return k(er)


def _sc_agg128(tab, er):
    @functools.partial(
        pl.kernel,
        out_type=jax.ShapeDtypeStruct((2, _NP, 128), jnp.float32),
        mesh=_MESH,
        scratch_types=[
            pltpu.VMEM((_JB, _BK), jnp.int32),
            pltpu.VMEM((_JB, _BK), jnp.int32),
            pltpu.VMEM((_BK, 128), jnp.float32),
            pltpu.VMEM((_BK, 128), jnp.float32),
            pltpu.VMEM((_ZR, 128), jnp.float32),
            pltpu.VMEM_SHARED((_NP, 128), jnp.float32),
            pltpu.SemaphoreType.DMA,
            pltpu.SemaphoreType.DMA,
        ],
    )
    def k(tab_hbm, er_hbm, out_hbm, src_v, dst_v, rows_a, rows_b, zrow_v,
          acc, sem_a, sem_b):
        c = lax.axis_index("c")
        s = lax.axis_index("s")
        zero = jnp.zeros((16,), jnp.float32)

        def fzero(i, carry):
            def fz2(m, carry2):
                zrow_v[i, pl.ds(m * 16, 16)] = zero
                return carry2

            lax.fori_loop(0, 8, fz2, 0)
            return carry

        lax.fori_loop(0, _ZR, fzero, 0)
        for t in range(_NZ):
            pltpu.sync_copy(zrow_v, acc.at[pl.ds(s * _RPS + t * _ZR, _ZR)])
        plsc.subcore_barrier()

        def start_gather(j, buf, sem):
            pltpu.async_copy(tab_hbm.at[c].at[src_v.at[j]], buf, sem)

        def wait_gather(buf, sem):
            pltpu.make_async_copy(tab_hbm.at[c].at[src_v.at[0]], buf,
                                  sem).wait()

        def chunk(g, carry):
            pltpu.sync_copy(er_hbm.at[0, s, g], src_v)
            pltpu.sync_copy(er_hbm.at[1, s, g], dst_v)
            start_gather(0, rows_a, sem_a)

            def pair(p, carry2):
                j = p * 2
                wait_gather(rows_a, sem_a)
                start_gather(j + 1, rows_b, sem_b)
                pltpu.sync_copy(rows_a, acc.at[dst_v.at[j]], add=True)
                wait_gather(rows_b, sem_b)
                start_gather(j + 2, rows_a, sem_a)
                pltpu.sync_copy(rows_b, acc.at[dst_v.at[j + 1]], add=True)
                return carry2

            lax.fori_loop(0, (_JB - 1) // 2, pair, 0)
            wait_gather(rows_a, sem_a)
            pltpu.sync_copy(rows_a, acc.at[dst_v.at[_JB - 1]], add=True)
            return carry

        lax.fori_loop(0, _NCK, chunk, 0)
        plsc.subcore_barrier()
        pltpu.sync_copy(acc.at[pl.ds(s * _RPS, _RPS)],
                        out_hbm.at[c, pl.ds(s * _RPS, _RPS)])

    return k(tab, er)


def _sc_scol(st1, er1):
    @functools.partial(
        pl.kernel,
        out_type=jax.ShapeDtypeStruct((_NS * _NP,), jnp.float32),
        mesh=_MESH,
        compiler_params=pltpu.CompilerParams(needs_layout_passes=False),
        scratch_types=[
            pltpu.VMEM((_EP,), jnp.int32),
            pltpu.VMEM((_EP,), jnp.int32),
            pltpu.VMEM((_NP,), jnp.float32),
            pltpu.VMEM((_NP,), jnp.float32),
        ],
    )
    def k(st_hbm, er_hbm, outs_hbm, src_v, dst_v, stcol_v, sacc_v):
        c = lax.axis_index("c")
        s = lax.axis_index("s")
        zero = jnp.zeros((16,), jnp.float32)

        @pl.when(c == 0)
        def _():
            pltpu.sync_copy(st_hbm, stcol_v)
            pltpu.sync_copy(er_hbm.at[pl.ds(s * _EP, _EP)], src_v)
            pltpu.sync_copy(er_hbm.at[pl.ds(_E + s * _EP, _EP)], dst_v)

            def fzs(i, carry):
                sacc_v[pl.ds(i * 16, 16)] = zero
                return carry

            lax.fori_loop(0, _NP // 16, fzs, 0)

            def body(t, carry):
                i16 = src_v[pl.ds(t * 16, 16)]
                d16 = dst_v[pl.ds(t * 16, 16)]
                v16 = plsc.load_gather(stcol_v, [i16])
                plsc.addupdate_scatter(sacc_v, [d16], v16)
                return carry

            lax.fori_loop(0, _EP // 16, body, 0)
            pltpu.sync_copy(sacc_v, outs_hbm.at[pl.ds(s * _NP, _NP)])

    return k(st1, er1)


def _tc_prep_body(x_ref, dego_ref, out_ref):
    d = dego_ref[:, 0:1]
    ns = jnp.where(d > 0, lax.rsqrt(d), 0.0)
    xn = x_ref[...] * ns
    out_ref[0] = xn[:, :128]
    out_ref[1] = xn[:, 128:]


def _tc_prep(x, dego):
    bn = 1000
    return pl.pallas_call(
        _tc_prep_body,
        grid=(_N // bn,),
        in_specs=[
            pl.BlockSpec((bn, 256), lambda i: (i, 0)),
            pl.BlockSpec((bn, 128), lambda i: (i, 0)),
        ],
        out_specs=pl.BlockSpec((2, bn, 128), lambda i: (0, i, 0)),
        out_shape=jax.ShapeDtypeStruct((2, _N, 128), jnp.float32),
    )(x, dego)


def _tc_mid_body(agg1_ref, degi_ref, dego_ref, We1_ref, be1_ref, Wg1_ref,
                 bg1_ref, We2_ref, Wfc_ref, ct_ref, st_ref):
    a = jnp.concatenate([agg1_ref[0], agg1_ref[1]], axis=1)
    di = degi_ref[:, 0:1]
    nd = jnp.where(di > 0, lax.rsqrt(di), 0.0)
    do = dego_ref[:, 0:1]
    no = jnp.where(do > 0, lax.rsqrt(do), 0.0)
    a = a * nd
    h1 = jax.nn.relu(jnp.dot(a, We1_ref[...], precision=_HI) + be1_ref[...])
    z1 = jax.nn.relu(jnp.dot(a, Wg1_ref[...], precision=_HI) + bg1_ref[...])
    wv = jnp.dot(We2_ref[...], Wfc_ref[...], precision=_HI)
    sc = jnp.dot(h1, wv, precision=_HI)
    z1n = z1 * no
    ct_ref[0] = z1n[:, :128]
    ct_ref[1] = z1n[:, 128:]
    st_ref[...] = (sc[:, 0:1] * no).reshape(_BM // 128, 128)


def _tc_mid(agg1, degi, dego, We1, be1r, Wg1, bg1r, We2, Wfc):
    full = lambda shape: pl.BlockSpec(shape, lambda i: tuple(0 for _ in shape))
    return pl.pallas_call(
        _tc_mid_body,
        grid=(_NP // _BM,),
        in_specs=[
            pl.BlockSpec((2, _BM, 128), lambda i: (0, i, 0)),
            pl.BlockSpec((_BM, 128), lambda i: (i, 0)),
            pl.BlockSpec((_BM, 128), lambda i: (i, 0)),
            full((256, 256)),
            full((1, 256)),
            full((256, 256)),
            full((1, 256)),
            full((256, 256)),
            full((256, 1)),
        ],
        out_specs=(
            pl.BlockSpec((2, _BM, 128), lambda i: (0, i, 0)),
            pl.BlockSpec((_BM // 128, 128), lambda i: (i, 0)),
        ),
        out_shape=(
            jax.ShapeDtypeStruct((2, _NP, 128), jnp.float32),
            jax.ShapeDtypeStruct((_SR, 128), jnp.float32),
        ),
    )(agg1, degi, dego, We1, be1r, Wg1, bg1r, We2, Wfc)


def _tc_final_body(agg2_ref, part_ref, degi_ref, degf_ref, Wg2_ref, bg2_ref,
                   Wc_ref, bc_ref, be2_ref, Wfc_ref, bfc_ref, y_ref, sf_ref,
                   z_ref):
    q = jnp.concatenate([agg2_ref[0], agg2_ref[1]], axis=1)
    di = degi_ref[:, 0:1]
    nd = jnp.where(di > 0, lax.rsqrt(di), 0.0)
    z = jnp.dot(q * nd, Wg2_ref[...], precision=_HI) + bg2_ref[...]
    y = jnp.dot(z, Wc_ref[...], precision=_HI) + bc_ref[...]
    sconst = jnp.dot(be2_ref[...], Wfc_ref[...], precision=_HI) + bfc_ref[...]
    df = degf_ref[...]
    ndf = jnp.where(df > 0, lax.rsqrt(df), 0.0)
    sagg = jnp.sum(part_ref[...], axis=0)
    sf_ref[...] = sagg * ndf + sconst
    y_ref[...] = y
    z_ref[...] = z


def _tc_final(agg2, parts, degi, degf, Wg2, bg2r, Wc, bcr, be2r, Wfc, bfcr):
    full = lambda shape: pl.BlockSpec(shape, lambda i: tuple(0 for _ in shape))
    return pl.pallas_call(
        _tc_final_body,
        grid=(_NP // _BM,),
        in_specs=[
            pl.BlockSpec((2, _BM, 128), lambda i: (0, i, 0)),
            pl.BlockSpec((_NS, _BM // 128, 128), lambda i: (0, i, 0)),
            pl.BlockSpec((_BM, 128), lambda i: (i, 0)),
            pl.BlockSpec((_BM // 128, 128), lambda i: (i, 0)),
            full((256, 256)),
            full((1, 256)),
            full((256, 1)),
            full((1, 1)),
            full((1, 256)),
            full((256, 1)),
            full((1, 1)),
        ],
        out_specs=(
            pl.BlockSpec((_BM, 1), lambda i: (i, 0)),
            pl.BlockSpec((_BM // 128, 128), lambda i: (i, 0)),
            pl.BlockSpec((_BM, 256), lambda i: (i, 0)),
        ),
        out_shape=(
            jax.ShapeDtypeStruct((_N, 1), jnp.float32),
            jax.ShapeDtypeStruct((_NP // 128, 128), jnp.float32),
            jax.ShapeDtypeStruct((_N, 256), jnp.float32),
        ),
    )(agg2, parts, degi, degf, Wg2, bg2r, Wc, bcr, be2r, Wfc, bfcr)


def kernel(x, edge_index, We1, be1, We2, be2, Wfc, bfc, Wg1, bg1, Wg2, bg2,
           Wc, bc):
    er = edge_index.reshape(2, _NS, _NCK, _JB, _BK)
    deg = _sc_degrees(er)
    dego = deg[0]
    degi = deg[1]
    xnt = _tc_prep(x, dego)
    agg1 = _sc_agg128(xnt, er)
    ct, st = _tc_mid(agg1, degi, dego, We1, be1.reshape(1, -1), Wg1,
                     bg1.reshape(1, -1), We2, Wfc)
    agg2 = _sc_agg128(ct, er)
    sparts = _sc_scol(st.reshape(-1), edge_index.reshape(-1))
    parts = sparts.reshape(_NS, _NP // 128, 128)
    degf = degi[:, 0].reshape(_NP // 128, 128)
    y, sf, z = _tc_final(agg2, parts, degi, degf, Wg2, bg2.reshape(1, -1), Wc,
                         bc.reshape(1, 1), be2.reshape(1, -1), Wfc,
                         bfc.reshape(1, 1))
    s = sf.reshape(-1)[:_N].reshape(_N, 1)
    return (y, s, z)


# speedup vs baseline: 6.5016x; 1.1250x over previous
import functools

import jax
import jax.numpy as jnp
from jax import lax
from jax.experimental import pallas as pl
from jax.experimental.pallas import tpu as pltpu
from jax.experimental.pallas import tpu_sc as plsc

_N = 10000
_E = 160000
_NS = 16
_EP = _E // _NS
_BK = 80
_NCK = 5
_JB = 25
_NP = 10240
_RPS = _NP // _NS
_ZR = 32
_NZ = _RPS // _ZR
_SR = _NP // 128
_BM = 1024
_HI = jax.lax.Precision.HIGHEST

_MESH = plsc.VectorSubcoreMesh(core_axis_name="c", subcore_axis_name="s")


def _sc_degrees(er1):
    @functools.partial(
        pl.kernel,
        out_type=(
            jax.ShapeDtypeStruct((_NP * 16,), jnp.float32),
            jax.ShapeDtypeStruct((_NP * 16,), jnp.float32),
        ),
        mesh=_MESH,
        compiler_params=pltpu.CompilerParams(needs_layout_passes=False),
        scratch_types=[
            pltpu.VMEM((_EP,), jnp.int32),
            pltpu.VMEM((_NP,), jnp.float32),
            pltpu.VMEM((16 * _RPS,), jnp.float32),
            pltpu.VMEM((16 * _RPS,), jnp.float32),
            pltpu.VMEM_SHARED((_NS * _NP,), jnp.float32),
        ],
    )
    def k(er_hbm, dego_hbm, degi_hbm, idx_v, acc_v, part_v, stage_v, slab):
        c = lax.axis_index("c")
        s = lax.axis_index("s")
        zero = jnp.zeros((16,), jnp.float32)
        one = jnp.ones((16,), jnp.float32)
        iota16 = lax.iota(jnp.int32, 16)

        def fz(i, carry):
            acc_v[pl.ds(i * 16, 16)] = zero
            return carry

        lax.fori_loop(0, _NP // 16, fz, 0)

        pltpu.sync_copy(er_hbm.at[pl.ds(c * _E + s * _EP, _EP)], idx_v)

        def body(t, carry):
            i16 = idx_v[pl.ds(t * 16, 16)]
            plsc.addupdate_scatter(acc_v, [i16], one)
            return carry

        lax.fori_loop(0, _EP // 16, body, 0)

        pltpu.sync_copy(acc_v, slab.at[pl.ds(s * _NP, _NP)])
        plsc.subcore_barrier()

        for p in range(_NS):
            pltpu.sync_copy(slab.at[pl.ds(p * _NP + s * _RPS, _RPS)],
                            part_v.at[pl.ds(p * _RPS, _RPS)])

        def red(w, carry):
            tot = part_v[pl.ds(w * 16, 16)]
            for p in range(1, _NS):
                tot = tot + part_v[pl.ds(p * _RPS + w * 16, 16)]
            n16 = iota16 + w * 16
            plsc.store_scatter(stage_v, [n16 * 16], tot)
            return carry

        lax.fori_loop(0, _RPS // 16, red, 0)

        @pl.when(c == 0)
        def _():
            pltpu.sync_copy(stage_v, dego_hbm.at[pl.ds(s * _RPS * 16,
                                                       _RPS * 16)])

        @pl.when(c == 1)
        def _():
            pltpu.sync_copy(stage_v, degi_hbm.at[pl.ds(s * _RPS * 16,
                                                       _RPS * 16)])

    return k(er1)


def _sc_agg128(tab, er):
    @functools.partial(
        pl.kernel,
        out_type=jax.ShapeDtypeStruct((2, _NP, 128), jnp.float32),
        mesh=_MESH,
        scratch_types=[
            pltpu.VMEM((_JB, _BK), jnp.int32),
            pltpu.VMEM((_JB, _BK), jnp.int32),
            pltpu.VMEM((_BK, 128), jnp.float32),
            pltpu.VMEM((_BK, 128), jnp.float32),
            pltpu.VMEM((_ZR, 128), jnp.float32),
            pltpu.VMEM_SHARED((_NP, 128), jnp.float32),
            pltpu.SemaphoreType.DMA,
            pltpu.SemaphoreType.DMA,
        ],
    )
    def k(tab_hbm, er_hbm, out_hbm, src_v, dst_v, rows_a, rows_b, zrow_v,
          acc, sem_a, sem_b):
        c = lax.axis_index("c")
        s = lax.axis_index("s")
        zero = jnp.zeros((16,), jnp.float32)

        def fzero(i, carry):
            def fz2(m, carry2):
                zrow_v[i, pl.ds(m * 16, 16)] = zero
                return carry2

            lax.fori_loop(0, 8, fz2, 0)
            return carry

        lax.fori_loop(0, _ZR, fzero, 0)
        for t in range(_NZ):
            pltpu.sync_copy(zrow_v, acc.at[pl.ds(s * _RPS + t * _ZR, _ZR)])
        plsc.subcore_barrier()

        def start_gather(j, buf, sem):
            pltpu.async_copy(tab_hbm.at[c].at[src_v.at[j]], buf, sem)

        def wait_gather(buf, sem):
            pltpu.make_async_copy(tab_hbm.at[c].at[src_v.at[0]], buf,
                                  sem).wait()

        def chunk(g, carry):
            pltpu.sync_copy(er_hbm.at[0, s, g], src_v)
            pltpu.sync_copy(er_hbm.at[1, s, g], dst_v)
            start_gather(0, rows_a, sem_a)

            def pair(p, carry2):
                j = p * 2
                wait_gather(rows_a, sem_a)
                start_gather(j + 1, rows_b, sem_b)
                pltpu.sync_copy(rows_a, acc.at[dst_v.at[j]], add=True)
                wait_gather(rows_b, sem_b)
                start_gather(j + 2, rows_a, sem_a)
                pltpu.sync_copy(rows_b, acc.at[dst_v.at[j + 1]], add=True)
                return carry2

            lax.fori_loop(0, (_JB - 1) // 2, pair, 0)
            wait_gather(rows_a, sem_a)
            pltpu.sync_copy(rows_a, acc.at[dst_v.at[_JB - 1]], add=True)
            return carry

        lax.fori_loop(0, _NCK, chunk, 0)
        plsc.subcore_barrier()
        pltpu.sync_copy(acc.at[pl.ds(s * _RPS, _RPS)],
                        out_hbm.at[c, pl.ds(s * _RPS, _RPS)])

    return k(tab, er)


def _sc_scol(st1, er1):
    @functools.partial(
        pl.kernel,
        out_type=jax.ShapeDtypeStruct((_NS * _NP,), jnp.float32),
        mesh=_MESH,
        compiler_params=pltpu.CompilerParams(needs_layout_passes=False),
        scratch_types=[
            pltpu.VMEM((_EP,), jnp.int32),
            pltpu.VMEM((_EP,), jnp.int32),
            pltpu.VMEM((_NP,), jnp.float32),
            pltpu.VMEM((_NP,), jnp.float32),
        ],
    )
    def k(st_hbm, er_hbm, outs_hbm, src_v, dst_v, stcol_v, sacc_v):
        c = lax.axis_index("c")
        s = lax.axis_index("s")
        zero = jnp.zeros((16,), jnp.float32)

        @pl.when(c == 0)
        def _():
            pltpu.sync_copy(st_hbm, stcol_v)
            pltpu.sync_copy(er_hbm.at[pl.ds(s * _EP, _EP)], src_v)
            pltpu.sync_copy(er_hbm.at[pl.ds(_E + s * _EP, _EP)], dst_v)

            def fzs(i, carry):
                sacc_v[pl.ds(i * 16, 16)] = zero
                return carry

            lax.fori_loop(0, _NP // 16, fzs, 0)

            def body(t, carry):
                i16 = src_v[pl.ds(t * 16, 16)]
                d16 = dst_v[pl.ds(t * 16, 16)]
                v16 = plsc.load_gather(stcol_v, [i16])
                plsc.addupdate_scatter(sacc_v, [d16], v16)
                return carry

            lax.fori_loop(0, _EP // 16, body, 0)
            pltpu.sync_copy(sacc_v, outs_hbm.at[pl.ds(s * _NP, _NP)])

    return k(st1, er1)


def _tc_prep_body(x_ref, dego_ref, out_ref):
    d = dego_ref[:, 0:1]
    ns = jnp.where(d > 0, lax.rsqrt(d), 0.0)
    xn = x_ref[...] * ns
    out_ref[0] = xn[:, :128]
    out_ref[1] = xn[:, 128:]


def _tc_prep(x, dego):
    bn = 1000
    return pl.pallas_call(
        _tc_prep_body,
        grid=(_N // bn,),
        in_specs=[
            pl.BlockSpec((bn, 256), lambda i: (i, 0)),
            pl.BlockSpec((bn, 16), lambda i: (i, 0)),
        ],
        out_specs=pl.BlockSpec((2, bn, 128), lambda i: (0, i, 0)),
        out_shape=jax.ShapeDtypeStruct((2, _N, 128), jnp.float32),
    )(x, dego)


def _tc_mid_body(agg1_ref, degi_ref, dego_ref, We1_ref, be1_ref, Wg1_ref,
                 bg1_ref, We2_ref, Wfc_ref, ct_ref, st_ref):
    a = jnp.concatenate([agg1_ref[0], agg1_ref[1]], axis=1)
    di = degi_ref[:, 0:1]
    nd = jnp.where(di > 0, lax.rsqrt(di), 0.0)
    do = dego_ref[:, 0:1]
    no = jnp.where(do > 0, lax.rsqrt(do), 0.0)
    a = a * nd
    h1 = jax.nn.relu(jnp.dot(a, We1_ref[...], precision=_HI) + be1_ref[...])
    z1 = jax.nn.relu(jnp.dot(a, Wg1_ref[...], precision=_HI) + bg1_ref[...])
    wv = jnp.dot(We2_ref[...], Wfc_ref[...], precision=_HI)
    sc = jnp.dot(h1, wv, precision=_HI)
    z1n = z1 * no
    ct_ref[0] = z1n[:, :128]
    ct_ref[1] = z1n[:, 128:]
    st_ref[...] = (sc[:, 0:1] * no).reshape(_BM // 128, 128)


def _tc_mid(agg1, degi, dego, We1, be1r, Wg1, bg1r, We2, Wfc):
    full = lambda shape: pl.BlockSpec(shape, lambda i: tuple(0 for _ in shape))
    return pl.pallas_call(
        _tc_mid_body,
        grid=(_NP // _BM,),
        in_specs=[
            pl.BlockSpec((2, _BM, 128), lambda i: (0, i, 0)),
            pl.BlockSpec((_BM, 16), lambda i: (i, 0)),
            pl.BlockSpec((_BM, 16), lambda i: (i, 0)),
            full((256, 256)),
            full((1, 256)),
            full((256, 256)),
            full((1, 256)),
            full((256, 256)),
            full((256, 1)),
        ],
        out_specs=(
            pl.BlockSpec((2, _BM, 128), lambda i: (0, i, 0)),
            pl.BlockSpec((_BM // 128, 128), lambda i: (i, 0)),
        ),
        out_shape=(
            jax.ShapeDtypeStruct((2, _NP, 128), jnp.float32),
            jax.ShapeDtypeStruct((_SR, 128), jnp.float32),
        ),
    )(agg1, degi, dego, We1, be1r, Wg1, bg1r, We2, Wfc)


def _tc_final_body(agg2_ref, part_ref, degi_ref, degf_ref, Wg2_ref, bg2_ref,
                   Wc_ref, bc_ref, be2_ref, Wfc_ref, bfc_ref, y_ref, sf_ref,
                   z_ref):
    q = jnp.concatenate([agg2_ref[0], agg2_ref[1]], axis=1)
    di = degi_ref[:, 0:1]
    nd = jnp.where(di > 0, lax.rsqrt(di), 0.0)
    z = jnp.dot(q * nd, Wg2_ref[...], precision=_HI) + bg2_ref[...]
    y = jnp.dot(z, Wc_ref[...], precision=_HI) + bc_ref[...]
    sconst = jnp.dot(be2_ref[...], Wfc_ref[...], precision=_HI) + bfc_ref[...]
    df = degf_ref[...]
    ndf = jnp.where(df > 0, lax.rsqrt(df), 0.0)
    sagg = jnp.sum(part_ref[...], axis=0)
    sf_ref[...] = sagg * ndf + sconst
    y_ref[...] = y
    z_ref[...] = z


def _tc_final(agg2, parts, degi, degf, Wg2, bg2r, Wc, bcr, be2r, Wfc, bfcr):
    full = lambda shape: pl.BlockSpec(shape, lambda i: tuple(0 for _ in shape))
    return pl.pallas_call(
        _tc_final_body,
        grid=(_NP // _BM,),
        in_specs=[
            pl.BlockSpec((2, _BM, 128), lambda i: (0, i, 0)),
            pl.BlockSpec((_NS, _BM // 128, 128), lambda i: (0, i, 0)),
            pl.BlockSpec((_BM, 16), lambda i: (i, 0)),
            pl.BlockSpec((_BM // 128, 128), lambda i: (i, 0)),
            full((256, 256)),
            full((1, 256)),
            full((256, 1)),
            full((1, 1)),
            full((1, 256)),
            full((256, 1)),
            full((1, 1)),
        ],
        out_specs=(
            pl.BlockSpec((_BM, 1), lambda i: (i, 0)),
            pl.BlockSpec((_BM // 128, 128), lambda i: (i, 0)),
            pl.BlockSpec((_BM, 256), lambda i: (i, 0)),
        ),
        out_shape=(
            jax.ShapeDtypeStruct((_N, 1), jnp.float32),
            jax.ShapeDtypeStruct((_NP // 128, 128), jnp.float32),
            jax.ShapeDtypeStruct((_N, 256), jnp.float32),
        ),
    )(agg2, parts, degi, degf, Wg2, bg2r, Wc, bcr, be2r, Wfc, bfcr)


def kernel(x, edge_index, We1, be1, We2, be2, Wfc, bfc, Wg1, bg1, Wg2, bg2,
           Wc, bc):
    er = edge_index.reshape(2, _NS, _NCK, _JB, _BK)
    dego_f, degi_f = _sc_degrees(edge_index.reshape(-1))
    dego = dego_f.reshape(_NP, 16)
    degi = degi_f.reshape(_NP, 16)
    xnt = _tc_prep(x, dego)
    agg1 = _sc_agg128(xnt, er)
    ct, st = _tc_mid(agg1, degi, dego, We1, be1.reshape(1, -1), Wg1,
                     bg1.reshape(1, -1), We2, Wfc)
    agg2 = _sc_agg128(ct, er)
    sparts = _sc_scol(st.reshape(-1), edge_index.reshape(-1))
    parts = sparts.reshape(_NS, _NP // 128, 128)
    degf = degi[:, 0].reshape(_NP // 128, 128)
    y, sf, z = _tc_final(agg2, parts, degi, degf, Wg2, bg2.reshape(1, -1), Wc,
                         bc.reshape(1, 1), be2.reshape(1, -1), Wfc,
                         bfc.reshape(1, 1))
    s = sf.reshape(-1)[:_N].reshape(_N, 1)
    return (y, s, z)
